# Initial kernel scaffold; baseline (speedup 1.0000x reference)
import functools
import math

import jax
import jax.numpy as jnp
from jax import lax
from jax.experimental import pallas as pl
from jax.experimental.pallas import tpu as pltpu
from jax.experimental.pallas import tpu_sc as plsc

N = 10000
E = 320000
H = 128
NG = 16
NF = 64
G = 64
NS = 1
CUTOFF = 10.0
LOG2 = math.log(2.0)


_BE = 3200
_BN = 1000


def _ssp(x):
    return jnp.maximum(x, 0.0) + jnp.log(1.0 + jnp.exp(-jnp.abs(x))) - LOG2


def _edge_filter_body(ea_ref, ew_ref, m1w0_ref, m1b0_ref, m2w0_ref, m2b0_ref,
                      m1w1_ref, m1b1_ref, m2w1_ref, m2b1_ref,
                      we0_ref, we1_ref):
    ea = ea_ref[...]
    c = 0.5 * (jnp.cos(ew_ref[...] * (math.pi / CUTOFF)) + 1.0)
    z0 = _ssp(jnp.dot(ea, m1w0_ref[...], preferred_element_type=jnp.float32)
              + m1b0_ref[...])
    we0_ref[...] = (jnp.dot(z0, m2w0_ref[...], preferred_element_type=jnp.float32)
                    + m2b0_ref[...]) * c
    z1 = _ssp(jnp.dot(ea, m1w1_ref[...], preferred_element_type=jnp.float32)
              + m1b1_ref[...])
    we1_ref[...] = (jnp.dot(z1, m2w1_ref[...], preferred_element_type=jnp.float32)
                    + m2b1_ref[...]) * c


def _edge_filter(ea, ew2, m1w0, m1b0, m2w0, m2b0, m1w1, m1b1, m2w1, m2b1):
    grid = (E // _BE,)
    wspec = pl.BlockSpec((NG, NF), lambda i: (0, 0))
    w2spec = pl.BlockSpec((NF, NF), lambda i: (0, 0))
    bspec = pl.BlockSpec((1, NF), lambda i: (0, 0))
    return pl.pallas_call(
        _edge_filter_body,
        grid=grid,
        in_specs=[
            pl.BlockSpec((_BE, NG), lambda i: (i, 0)),
            pl.BlockSpec((_BE, 1), lambda i: (i, 0)),
            wspec, bspec, w2spec, bspec,
            wspec, bspec, w2spec, bspec,
        ],
        out_specs=[
            pl.BlockSpec((_BE, NF), lambda i: (i, 0)),
            pl.BlockSpec((_BE, NF), lambda i: (i, 0)),
        ],
        out_shape=[
            jax.ShapeDtypeStruct((E, NF), jnp.float32),
            jax.ShapeDtypeStruct((E, NF), jnp.float32),
        ],
        compiler_params=pltpu.CompilerParams(
            dimension_semantics=("arbitrary",),
        ),
    )(ea, ew2, m1w0, m1b0, m2w0, m2b0, m1w1, m1b1, m2w1, m2b1)


def _node_pre_body(h_ref, ph_ref, pc_ref, b_ref, a_ref, v_ref, l1b_ref,
                   c1w_ref, h1_ref, xl_ref):
    sa = (jnp.sum(ph_ref[...], axis=1, keepdims=True)
          / jnp.maximum(pc_ref[...], 1.0))
    onehot = (b_ref[...] ==
              lax.broadcasted_iota(jnp.int32, (_BN, G), 1)).astype(jnp.float32)
    s = jnp.dot(onehot, sa, preferred_element_type=jnp.float32)
    h1 = (jnp.dot(h_ref[...], a_ref[...], preferred_element_type=jnp.float32)
          + s * v_ref[...] + l1b_ref[...])
    h1_ref[...] = h1
    xl_ref[...] = jnp.dot(h1, c1w_ref[...], preferred_element_type=jnp.float32)


def _node_pre(h, ph, pc, b2, a_t, v, l1b, c1w_t):
    grid = (N // _BN,)
    return pl.pallas_call(
        _node_pre_body,
        grid=grid,
        in_specs=[
            pl.BlockSpec((_BN, H), lambda i: (i, 0)),
            pl.BlockSpec((G, H), lambda i: (0, 0)),
            pl.BlockSpec((G, 1), lambda i: (0, 0)),
            pl.BlockSpec((_BN, 1), lambda i: (i, 0)),
            pl.BlockSpec((H, H), lambda i: (0, 0)),
            pl.BlockSpec((1, H), lambda i: (0, 0)),
            pl.BlockSpec((1, H), lambda i: (0, 0)),
            pl.BlockSpec((H, NF), lambda i: (0, 0)),
        ],
        out_specs=[
            pl.BlockSpec((_BN, H), lambda i: (i, 0)),
            pl.BlockSpec((_BN, NF), lambda i: (i, 0)),
        ],
        out_shape=[
            jax.ShapeDtypeStruct((N, H), jnp.float32),
            jax.ShapeDtypeStruct((N, NF), jnp.float32),
        ],
        compiler_params=pltpu.CompilerParams(
            dimension_semantics=("arbitrary",),
        ),
    )(h, ph, pc, b2, a_t, v, l1b, c1w_t)


def _node_post_pool_body(a0_ref, a1_ref, h1_ref, b_ref, c2w_ref, c2b_ref,
                         ilw_ref, ilb_ref, h2_ref, ph_ref, pc_ref):
    agg = a0_ref[...] + a1_ref[...]
    t = _ssp(jnp.dot(agg, c2w_ref[...], preferred_element_type=jnp.float32)
             + c2b_ref[...])
    h2 = (h1_ref[...]
          + jnp.dot(t, ilw_ref[...], preferred_element_type=jnp.float32)
          + ilb_ref[...])
    h2_ref[...] = h2
    onehot = (b_ref[...] ==
              lax.broadcasted_iota(jnp.int32, (_BN, G), 1)).astype(jnp.float32)
    ph = lax.dot_general(onehot, h2, (((0,), (0,)), ((), ())),
                         preferred_element_type=jnp.float32)
    pc = jnp.sum(onehot, axis=0)[:, None]
    i = pl.program_id(0)

    @pl.when(i == 0)
    def _():
        ph_ref[...] = ph
        pc_ref[...] = pc

    @pl.when(i != 0)
    def _():
        ph_ref[...] += ph
        pc_ref[...] += pc


def _node_post_pool(agg2, h1, b2, c2w_t, c2b, ilw_t, ilb):
    grid = (N // _BN,)
    nb = N // _BN
    return pl.pallas_call(
        _node_post_pool_body,
        grid=grid,
        in_specs=[
            pl.BlockSpec((_BN, NF), lambda i: (i, 0)),
            pl.BlockSpec((_BN, NF), lambda i, nb=nb: (i + nb, 0)),
            pl.BlockSpec((_BN, H), lambda i: (i, 0)),
            pl.BlockSpec((_BN, 1), lambda i: (i, 0)),
            pl.BlockSpec((NF, H), lambda i: (0, 0)),
            pl.BlockSpec((1, H), lambda i: (0, 0)),
            pl.BlockSpec((H, H), lambda i: (0, 0)),
            pl.BlockSpec((1, H), lambda i: (0, 0)),
        ],
        out_specs=[
            pl.BlockSpec((_BN, H), lambda i: (i, 0)),
            pl.BlockSpec((G, H), lambda i: (0, 0)),
            pl.BlockSpec((G, 1), lambda i: (0, 0)),
        ],
        out_shape=[
            jax.ShapeDtypeStruct((N, H), jnp.float32),
            jax.ShapeDtypeStruct((G, H), jnp.float32),
            jax.ShapeDtypeStruct((G, 1), jnp.float32),
        ],
        compiler_params=pltpu.CompilerParams(
            dimension_semantics=("arbitrary",),
        ),
    )(agg2, agg2, h1, b2, c2w_t, c2b, ilw_t, ilb)


def _node_post_final_body(a0_ref, a1_ref, h1_ref, c2w_ref, c2b_ref,
                          ilw_ref, ilb_ref, ow_ref, ob_ref, out_ref):
    agg = a0_ref[...] + a1_ref[...]
    t = _ssp(jnp.dot(agg, c2w_ref[...], preferred_element_type=jnp.float32)
             + c2b_ref[...])
    h2 = (h1_ref[...]
          + jnp.dot(t, ilw_ref[...], preferred_element_type=jnp.float32)
          + ilb_ref[...])
    out_ref[...] = jnp.maximum(
        jnp.dot(h2, ow_ref[...], preferred_element_type=jnp.float32)
        + ob_ref[...], 0.0)


def _node_post_final(agg2, h1, c2w_t, c2b, ilw_t, ilb, ow_t, ob):
    grid = (N // _BN,)
    nb = N // _BN
    return pl.pallas_call(
        _node_post_final_body,
        grid=grid,
        in_specs=[
            pl.BlockSpec((_BN, NF), lambda i: (i, 0)),
            pl.BlockSpec((_BN, NF), lambda i, nb=nb: (i + nb, 0)),
            pl.BlockSpec((_BN, H), lambda i: (i, 0)),
            pl.BlockSpec((NF, H), lambda i: (0, 0)),
            pl.BlockSpec((1, H), lambda i: (0, 0)),
            pl.BlockSpec((H, H), lambda i: (0, 0)),
            pl.BlockSpec((1, H), lambda i: (0, 0)),
            pl.BlockSpec((H, NF), lambda i: (0, 0)),
            pl.BlockSpec((1, NF), lambda i: (0, 0)),
        ],
        out_specs=pl.BlockSpec((_BN, NF), lambda i: (i, 0)),
        out_shape=jax.ShapeDtypeStruct((N, NF), jnp.float32),
        compiler_params=pltpu.CompilerParams(
            dimension_semantics=("arbitrary",),
        ),
    )(agg2, agg2, h1, c2w_t, c2b, ilw_t, ilb, ow_t, ob)


_NC = 2
_NSUB = 16
_NW = _NC * _NSUB
_CH = 80
_NCHUNK = (E // _NW) // _CH
_RPW = N // _NSUB


def _edge_agg_body(xl_hbm, we_hbm, si_hbm, out_hbm,
                   idx_v, we_v, rows_v, acc_sh, sem):
    cid = lax.axis_index("c")
    sid = lax.axis_index("s")
    wid = sid * _NC + cid

    def _zrow(i, _):
        for j in range(NF // 16):
            rows_v[i, pl.ds(j * 16, 16)] = jnp.zeros((16,), jnp.float32)
        return 0
    lax.fori_loop(0, _CH, _zrow, 0)
    base_r = sid * _RPW
    done = 0
    while done < _RPW:
        n = min(_CH, _RPW - done)
        pltpu.sync_copy(rows_v.at[pl.ds(0, n)],
                        acc_sh.at[pl.ds(base_r + done, n)])
        done += n
    plsc.subcore_barrier()

    def _chunk(ci, _):
        base = wid * (E // _NW) + ci * _CH
        pltpu.sync_copy(si_hbm.at[:, pl.ds(base, _CH)], idx_v)
        cp_we = pltpu.async_copy(we_hbm.at[pl.ds(base, _CH)], we_v, sem)
        pltpu.async_copy(xl_hbm.at[idx_v.at[0]], rows_v, sem).wait()
        cp_we.wait()

        def _mul(i, _):
            for j in range(NF // 16):
                sl = pl.ds(j * 16, 16)
                rows_v[i, sl] = rows_v[i, sl] * we_v[i, sl]
            return 0
        lax.fori_loop(0, _CH, _mul, 0)
        pltpu.sync_copy(rows_v, acc_sh.at[idx_v.at[1]], add=True)
        return 0
    lax.fori_loop(0, _NCHUNK, _chunk, 0)

    plsc.subcore_barrier()
    done = 0
    while done < _RPW:
        n = min(4 * _CH, _RPW - done)
        pltpu.sync_copy(acc_sh.at[pl.ds(base_r + done, n)],
                        out_hbm.at[pl.ds(cid * N + base_r + done, n)])
        done += n


def _edge_agg(xl, we, si):
    mesh = plsc.VectorSubcoreMesh(core_axis_name="c", subcore_axis_name="s")
    f = pl.kernel(
        _edge_agg_body,
        out_type=jax.ShapeDtypeStruct((_NC * N, NF), jnp.float32),
        mesh=mesh,
        scratch_types=[
            pltpu.VMEM((2, _CH), jnp.int32),
            pltpu.VMEM((_CH, NF), jnp.float32),
            pltpu.VMEM((_CH, NF), jnp.float32),
            pltpu.VMEM_SHARED((N, NF), jnp.float32),
            pltpu.SemaphoreType.DMA,
        ],
    )
    return f(xl, we, si)


def kernel(h, edge_index, edge_weight, edge_attr, state_attr, batch,
           lin1_w_0, lin1_b_0, mlp1_w_0, mlp1_b_0, mlp2_w_0, mlp2_b_0,
           cf1_w_0, cf2_w_0, cf2_b_0, il_w_0, il_b_0,
           lin1_w_1, lin1_b_1, mlp1_w_1, mlp1_b_1, mlp2_w_1, mlp2_b_1,
           cf1_w_1, cf2_w_1, cf2_b_1, il_w_1, il_b_1, out_w, out_b):
    ew2 = edge_weight[:, None]
    b2 = batch[:, None]

    we0, we1 = _edge_filter(
        edge_attr, ew2,
        mlp1_w_0.T, mlp1_b_0[None, :], mlp2_w_0.T, mlp2_b_0[None, :],
        mlp1_w_1.T, mlp1_b_1[None, :], mlp2_w_1.T, mlp2_b_1[None, :])

    ph0 = jnp.pad(state_attr, ((0, 0), (0, H - NS)))
    pc0 = jnp.ones((G, 1), jnp.float32)

    h1, xl = _node_pre(h, ph0, pc0, b2,
                       lin1_w_0[:, NS:].T, lin1_w_0[:, :NS].T,
                       lin1_b_0[None, :], cf1_w_0.T)
    agg2 = _edge_agg(xl, we0, edge_index)
    h2, ph, pc = _node_post_pool(agg2, h1, b2, cf2_w_0.T, cf2_b_0[None, :],
                                 il_w_0.T, il_b_0[None, :])

    h3, xl1 = _node_pre(h2, ph, pc, b2,
                        lin1_w_1[:, NS:].T, lin1_w_1[:, :NS].T,
                        lin1_b_1[None, :], cf1_w_1.T)
    agg2b = _edge_agg(xl1, we1, edge_index)
    return _node_post_final(agg2b, h3, cf2_w_1.T, cf2_b_1[None, :],
                            il_w_1.T, il_b_1[None, :], out_w.T,
                            out_b[None, :])


# speedup vs baseline: 1.9053x; 1.9053x over previous
import functools
import math

import jax
import jax.numpy as jnp
from jax import lax
from jax.experimental import pallas as pl
from jax.experimental.pallas import tpu as pltpu
from jax.experimental.pallas import tpu_sc as plsc

N = 10000
E = 320000
H = 128
NG = 16
NF = 64
G = 64
NS = 1
CUTOFF = 10.0
LOG2 = math.log(2.0)


_BE = 3200
_BN = 1000


def _ssp(x):
    return jnp.maximum(x, 0.0) + jnp.log(1.0 + jnp.exp(-jnp.abs(x))) - LOG2


def _edge_filter_body(ea_ref, ew_ref, m1w0_ref, m1b0_ref, m2w0_ref, m2b0_ref,
                      m1w1_ref, m1b1_ref, m2w1_ref, m2b1_ref,
                      we0_ref, we1_ref):
    ea = ea_ref[...]
    c = 0.5 * (jnp.cos(ew_ref[...] * (math.pi / CUTOFF)) + 1.0)
    z0 = _ssp(jnp.dot(ea, m1w0_ref[...], preferred_element_type=jnp.float32)
              + m1b0_ref[...])
    we0_ref[...] = (jnp.dot(z0, m2w0_ref[...], preferred_element_type=jnp.float32)
                    + m2b0_ref[...]) * c
    z1 = _ssp(jnp.dot(ea, m1w1_ref[...], preferred_element_type=jnp.float32)
              + m1b1_ref[...])
    we1_ref[...] = (jnp.dot(z1, m2w1_ref[...], preferred_element_type=jnp.float32)
                    + m2b1_ref[...]) * c


def _edge_filter(ea, ew2, m1w0, m1b0, m2w0, m2b0, m1w1, m1b1, m2w1, m2b1):
    grid = (E // _BE,)
    wspec = pl.BlockSpec((NG, NF), lambda i: (0, 0))
    w2spec = pl.BlockSpec((NF, NF), lambda i: (0, 0))
    bspec = pl.BlockSpec((1, NF), lambda i: (0, 0))
    return pl.pallas_call(
        _edge_filter_body,
        grid=grid,
        in_specs=[
            pl.BlockSpec((_BE, NG), lambda i: (i, 0)),
            pl.BlockSpec((_BE, 1), lambda i: (i, 0)),
            wspec, bspec, w2spec, bspec,
            wspec, bspec, w2spec, bspec,
        ],
        out_specs=[
            pl.BlockSpec((_BE, NF), lambda i: (i, 0)),
            pl.BlockSpec((_BE, NF), lambda i: (i, 0)),
        ],
        out_shape=[
            jax.ShapeDtypeStruct((E, NF), jnp.float32),
            jax.ShapeDtypeStruct((E, NF), jnp.float32),
        ],
        compiler_params=pltpu.CompilerParams(
            dimension_semantics=("arbitrary",),
        ),
    )(ea, ew2, m1w0, m1b0, m2w0, m2b0, m1w1, m1b1, m2w1, m2b1)


def _node_pre_body(h_ref, ph_ref, pc_ref, b_ref, a_ref, v_ref, l1b_ref,
                   c1w_ref, h1_ref, xl_ref):
    sa = (jnp.sum(ph_ref[...], axis=1, keepdims=True)
          / jnp.maximum(pc_ref[...], 1.0))
    onehot = (b_ref[...] ==
              lax.broadcasted_iota(jnp.int32, (_BN, G), 1)).astype(jnp.float32)
    s = jnp.dot(onehot, sa, preferred_element_type=jnp.float32)
    h1 = (jnp.dot(h_ref[...], a_ref[...], preferred_element_type=jnp.float32)
          + s * v_ref[...] + l1b_ref[...])
    h1_ref[...] = h1
    xl_ref[...] = jnp.dot(h1, c1w_ref[...], preferred_element_type=jnp.float32)


def _node_pre(h, ph, pc, b2, a_t, v, l1b, c1w_t):
    grid = (N // _BN,)
    return pl.pallas_call(
        _node_pre_body,
        grid=grid,
        in_specs=[
            pl.BlockSpec((_BN, H), lambda i: (i, 0)),
            pl.BlockSpec((G, H), lambda i: (0, 0)),
            pl.BlockSpec((G, 1), lambda i: (0, 0)),
            pl.BlockSpec((_BN, 1), lambda i: (i, 0)),
            pl.BlockSpec((H, H), lambda i: (0, 0)),
            pl.BlockSpec((1, H), lambda i: (0, 0)),
            pl.BlockSpec((1, H), lambda i: (0, 0)),
            pl.BlockSpec((H, NF), lambda i: (0, 0)),
        ],
        out_specs=[
            pl.BlockSpec((_BN, H), lambda i: (i, 0)),
            pl.BlockSpec((_BN, NF), lambda i: (i, 0)),
        ],
        out_shape=[
            jax.ShapeDtypeStruct((N, H), jnp.float32),
            jax.ShapeDtypeStruct((N, NF), jnp.float32),
        ],
        compiler_params=pltpu.CompilerParams(
            dimension_semantics=("arbitrary",),
        ),
    )(h, ph, pc, b2, a_t, v, l1b, c1w_t)


def _node_post_pool_body(a0_ref, a1_ref, h1_ref, b_ref, c2w_ref, c2b_ref,
                         ilw_ref, ilb_ref, h2_ref, ph_ref, pc_ref):
    agg = a0_ref[...] + a1_ref[...]
    t = _ssp(jnp.dot(agg, c2w_ref[...], preferred_element_type=jnp.float32)
             + c2b_ref[...])
    h2 = (h1_ref[...]
          + jnp.dot(t, ilw_ref[...], preferred_element_type=jnp.float32)
          + ilb_ref[...])
    h2_ref[...] = h2
    onehot = (b_ref[...] ==
              lax.broadcasted_iota(jnp.int32, (_BN, G), 1)).astype(jnp.float32)
    ph = lax.dot_general(onehot, h2, (((0,), (0,)), ((), ())),
                         preferred_element_type=jnp.float32)
    pc = jnp.sum(onehot, axis=0)[:, None]
    i = pl.program_id(0)

    @pl.when(i == 0)
    def _():
        ph_ref[...] = ph
        pc_ref[...] = pc

    @pl.when(i != 0)
    def _():
        ph_ref[...] += ph
        pc_ref[...] += pc


def _node_post_pool(agg2, h1, b2, c2w_t, c2b, ilw_t, ilb):
    grid = (N // _BN,)
    nb = N // _BN
    return pl.pallas_call(
        _node_post_pool_body,
        grid=grid,
        in_specs=[
            pl.BlockSpec((_BN, NF), lambda i: (i, 0)),
            pl.BlockSpec((_BN, NF), lambda i, nb=nb: (i + nb, 0)),
            pl.BlockSpec((_BN, H), lambda i: (i, 0)),
            pl.BlockSpec((_BN, 1), lambda i: (i, 0)),
            pl.BlockSpec((NF, H), lambda i: (0, 0)),
            pl.BlockSpec((1, H), lambda i: (0, 0)),
            pl.BlockSpec((H, H), lambda i: (0, 0)),
            pl.BlockSpec((1, H), lambda i: (0, 0)),
        ],
        out_specs=[
            pl.BlockSpec((_BN, H), lambda i: (i, 0)),
            pl.BlockSpec((G, H), lambda i: (0, 0)),
            pl.BlockSpec((G, 1), lambda i: (0, 0)),
        ],
        out_shape=[
            jax.ShapeDtypeStruct((N, H), jnp.float32),
            jax.ShapeDtypeStruct((G, H), jnp.float32),
            jax.ShapeDtypeStruct((G, 1), jnp.float32),
        ],
        compiler_params=pltpu.CompilerParams(
            dimension_semantics=("arbitrary",),
        ),
    )(agg2, agg2, h1, b2, c2w_t, c2b, ilw_t, ilb)


def _node_post_final_body(a0_ref, a1_ref, h1_ref, c2w_ref, c2b_ref,
                          ilw_ref, ilb_ref, ow_ref, ob_ref, out_ref):
    agg = a0_ref[...] + a1_ref[...]
    t = _ssp(jnp.dot(agg, c2w_ref[...], preferred_element_type=jnp.float32)
             + c2b_ref[...])
    h2 = (h1_ref[...]
          + jnp.dot(t, ilw_ref[...], preferred_element_type=jnp.float32)
          + ilb_ref[...])
    out_ref[...] = jnp.maximum(
        jnp.dot(h2, ow_ref[...], preferred_element_type=jnp.float32)
        + ob_ref[...], 0.0)


def _node_post_final(agg2, h1, c2w_t, c2b, ilw_t, ilb, ow_t, ob):
    grid = (N // _BN,)
    nb = N // _BN
    return pl.pallas_call(
        _node_post_final_body,
        grid=grid,
        in_specs=[
            pl.BlockSpec((_BN, NF), lambda i: (i, 0)),
            pl.BlockSpec((_BN, NF), lambda i, nb=nb: (i + nb, 0)),
            pl.BlockSpec((_BN, H), lambda i: (i, 0)),
            pl.BlockSpec((NF, H), lambda i: (0, 0)),
            pl.BlockSpec((1, H), lambda i: (0, 0)),
            pl.BlockSpec((H, H), lambda i: (0, 0)),
            pl.BlockSpec((1, H), lambda i: (0, 0)),
            pl.BlockSpec((H, NF), lambda i: (0, 0)),
            pl.BlockSpec((1, NF), lambda i: (0, 0)),
        ],
        out_specs=pl.BlockSpec((_BN, NF), lambda i: (i, 0)),
        out_shape=jax.ShapeDtypeStruct((N, NF), jnp.float32),
        compiler_params=pltpu.CompilerParams(
            dimension_semantics=("arbitrary",),
        ),
    )(agg2, agg2, h1, c2w_t, c2b, ilw_t, ilb, ow_t, ob)


_NC = 2
_NSUB = 16
_NW = _NC * _NSUB
_CH = 80
_NCHUNK = (E // _NW) // _CH
_RPW = 624
_RREM = N - _NSUB * _RPW


def _edge_agg_body(xl_hbm, we_hbm, src_hbm, dst_hbm, out_hbm,
                   src_v, dst_v, we_v, rows_v, acc_sh, sem):
    cid = lax.axis_index("c")
    sid = lax.axis_index("s")
    wid = sid * _NC + cid

    def _zrow(i, _):
        for j in range(NF // 16):
            rows_v[i, pl.ds(j * 16, 16)] = jnp.zeros((16,), jnp.float32)
        return 0
    lax.fori_loop(0, _CH, _zrow, 0)
    base_r = sid * _RPW
    done = 0
    while done < _RPW:
        n = min(_CH, _RPW - done)
        pltpu.sync_copy(rows_v.at[pl.ds(0, n)],
                        acc_sh.at[pl.ds(base_r + done, n)])
        done += n

    @pl.when(sid == _NSUB - 1)
    def _():
        pltpu.sync_copy(rows_v.at[pl.ds(0, _RREM)],
                        acc_sh.at[pl.ds(_NSUB * _RPW, _RREM)])
    plsc.subcore_barrier()

    def _chunk(ci, _):
        base = wid * (E // _NW) + ci * _CH
        pltpu.sync_copy(src_hbm.at[pl.ds(base, _CH)], src_v)
        pltpu.sync_copy(dst_hbm.at[pl.ds(base, _CH)], dst_v)
        cp_we = pltpu.async_copy(we_hbm.at[pl.ds(base, _CH)], we_v, sem)
        pltpu.async_copy(xl_hbm.at[src_v], rows_v, sem).wait()
        cp_we.wait()

        def _mul(i, _):
            for j in range(NF // 16):
                sl = pl.ds(j * 16, 16)
                rows_v[i, sl] = rows_v[i, sl] * we_v[i, sl]
            return 0
        lax.fori_loop(0, _CH, _mul, 0)
        pltpu.sync_copy(rows_v, acc_sh.at[dst_v], add=True)
        return 0
    lax.fori_loop(0, _NCHUNK, _chunk, 0)

    plsc.subcore_barrier()
    pltpu.sync_copy(acc_sh.at[pl.ds(base_r, _RPW)],
                    out_hbm.at[pl.ds(cid * N + base_r, _RPW)])

    @pl.when(sid == _NSUB - 1)
    def _():
        pltpu.sync_copy(acc_sh.at[pl.ds(_NSUB * _RPW, _RREM)],
                        out_hbm.at[pl.ds(cid * N + _NSUB * _RPW, _RREM)])


def _edge_agg(xl, we, src, dst):
    mesh = plsc.VectorSubcoreMesh(core_axis_name="c", subcore_axis_name="s")
    f = pl.kernel(
        _edge_agg_body,
        out_type=jax.ShapeDtypeStruct((_NC * N, NF), jnp.float32),
        mesh=mesh,
        scratch_types=[
            pltpu.VMEM((_CH,), jnp.int32),
            pltpu.VMEM((_CH,), jnp.int32),
            pltpu.VMEM((_CH, NF), jnp.float32),
            pltpu.VMEM((_CH, NF), jnp.float32),
            pltpu.VMEM_SHARED((N, NF), jnp.float32),
            pltpu.SemaphoreType.DMA,
        ],
        compiler_params=pltpu.CompilerParams(use_tc_tiling_on_sc=False),
    )
    return f(xl, we, src, dst)


def kernel(h, edge_index, edge_weight, edge_attr, state_attr, batch,
           lin1_w_0, lin1_b_0, mlp1_w_0, mlp1_b_0, mlp2_w_0, mlp2_b_0,
           cf1_w_0, cf2_w_0, cf2_b_0, il_w_0, il_b_0,
           lin1_w_1, lin1_b_1, mlp1_w_1, mlp1_b_1, mlp2_w_1, mlp2_b_1,
           cf1_w_1, cf2_w_1, cf2_b_1, il_w_1, il_b_1, out_w, out_b):
    ew2 = edge_weight[:, None]
    b2 = batch[:, None]
    src = edge_index[0]
    dst = edge_index[1]

    we0, we1 = _edge_filter(
        edge_attr, ew2,
        mlp1_w_0.T, mlp1_b_0[None, :], mlp2_w_0.T, mlp2_b_0[None, :],
        mlp1_w_1.T, mlp1_b_1[None, :], mlp2_w_1.T, mlp2_b_1[None, :])

    ph0 = jnp.pad(state_attr, ((0, 0), (0, H - NS)))
    pc0 = jnp.ones((G, 1), jnp.float32)

    h1, xl = _node_pre(h, ph0, pc0, b2,
                       lin1_w_0[:, NS:].T, lin1_w_0[:, :NS].T,
                       lin1_b_0[None, :], cf1_w_0.T)
    agg2 = _edge_agg(xl, we0, src, dst)
    h2, ph, pc = _node_post_pool(agg2, h1, b2, cf2_w_0.T, cf2_b_0[None, :],
                                 il_w_0.T, il_b_0[None, :])

    h3, xl1 = _node_pre(h2, ph, pc, b2,
                        lin1_w_1[:, NS:].T, lin1_w_1[:, :NS].T,
                        lin1_b_1[None, :], cf1_w_1.T)
    agg2b = _edge_agg(xl1, we1, src, dst)
    return _node_post_final(agg2b, h3, cf2_w_1.T, cf2_b_1[None, :],
                            il_w_1.T, il_b_1[None, :], out_w.T,
                            out_b[None, :])


# speedup vs baseline: 4.3417x; 2.2788x over previous
import functools
import math

import jax
import jax.numpy as jnp
from jax import lax
from jax.experimental import pallas as pl
from jax.experimental.pallas import tpu as pltpu
from jax.experimental.pallas import tpu_sc as plsc

N = 10000
E = 320000
H = 128
NG = 16
NF = 64
G = 64
NS = 1
CUTOFF = 10.0
LOG2 = math.log(2.0)


_BE = 3072
_BN = 1000


def _ssp(x):
    return jnp.maximum(x, 0.0) + jnp.log(1.0 + jnp.exp(-jnp.abs(x))) - LOG2


def _cos_poly(t):
    u = t * t
    return 1.0 + u * (-0.5 + u * (1.0 / 24.0 + u * (-1.0 / 720.0
                                                    + u * (1.0 / 40320.0))))


def _edge_filter_body(ea_ref, ew_ref, m1w_ref, m1b_ref, m2w_ref, m2b_ref,
                      we0_ref, we1_ref):
    z = _ssp(jnp.dot(ea_ref[...], m1w_ref[...],
                     preferred_element_type=jnp.float32) + m1b_ref[...])
    z2 = (jnp.dot(z, m2w_ref[...], preferred_element_type=jnp.float32)
          + m2b_ref[...])
    c = 0.5 * (_cos_poly(ew_ref[...] * (math.pi / CUTOFF)) + 1.0)
    rows = lax.broadcasted_iota(jnp.int32, (_BE, 128), 0)
    asel = (rows // 128 ==
            lax.broadcasted_iota(jnp.int32, (_BE, 128), 1)).astype(jnp.float32)
    msel = (rows % 128 ==
            lax.broadcasted_iota(jnp.int32, (_BE, 128), 1)).astype(jnp.float32)
    cpad = jnp.concatenate(
        [c, jnp.zeros((128 - _BE // 128, 128), jnp.float32)], axis=0)
    c_sel = jnp.dot(asel, cpad, preferred_element_type=jnp.float32) * msel
    c_rep = jnp.dot(c_sel, jnp.ones((128, 2 * NF), jnp.float32),
                    preferred_element_type=jnp.float32)
    wec = z2 * c_rep
    we0_ref[...] = wec[:, :NF]
    we1_ref[...] = wec[:, NF:]


def _edge_filter(ea, ew_mat, m1w, m1b, m2w, m2b):
    grid = (pl.cdiv(E, _BE),)
    return pl.pallas_call(
        _edge_filter_body,
        grid=grid,
        in_specs=[
            pl.BlockSpec((_BE, NG), lambda i: (i, 0)),
            pl.BlockSpec((_BE // 128, 128), lambda i: (i, 0)),
            pl.BlockSpec((NG, 2 * NF), lambda i: (0, 0)),
            pl.BlockSpec((1, 2 * NF), lambda i: (0, 0)),
            pl.BlockSpec((2 * NF, 2 * NF), lambda i: (0, 0)),
            pl.BlockSpec((1, 2 * NF), lambda i: (0, 0)),
        ],
        out_specs=[
            pl.BlockSpec((_BE, NF), lambda i: (i, 0)),
            pl.BlockSpec((_BE, NF), lambda i: (i, 0)),
        ],
        out_shape=[
            jax.ShapeDtypeStruct((E, NF), jnp.float32),
            jax.ShapeDtypeStruct((E, NF), jnp.float32),
        ],
        compiler_params=pltpu.CompilerParams(
            dimension_semantics=("arbitrary",),
        ),
    )(ea, ew_mat, m1w, m1b, m2w, m2b)


def _node_pre_body(h_ref, ph_ref, pc_ref, b_ref, a_ref, v_ref, l1b_ref,
                   c1w_ref, h1_ref, xl_ref):
    sa = (jnp.sum(ph_ref[...], axis=1, keepdims=True)
          / jnp.maximum(pc_ref[...], 1.0))
    onehot = (b_ref[...] ==
              lax.broadcasted_iota(jnp.int32, (_BN, G), 1)).astype(jnp.float32)
    s = jnp.dot(onehot, sa, preferred_element_type=jnp.float32)
    h1 = (jnp.dot(h_ref[...], a_ref[...], preferred_element_type=jnp.float32)
          + s * v_ref[...] + l1b_ref[...])
    h1_ref[...] = h1
    xl_ref[...] = jnp.dot(h1, c1w_ref[...], preferred_element_type=jnp.float32)


def _node_pre(h, ph, pc, b2, a_t, v, l1b, c1w_t):
    grid = (N // _BN,)
    return pl.pallas_call(
        _node_pre_body,
        grid=grid,
        in_specs=[
            pl.BlockSpec((_BN, H), lambda i: (i, 0)),
            pl.BlockSpec((G, H), lambda i: (0, 0)),
            pl.BlockSpec((G, 1), lambda i: (0, 0)),
            pl.BlockSpec((_BN, 1), lambda i: (i, 0)),
            pl.BlockSpec((H, H), lambda i: (0, 0)),
            pl.BlockSpec((1, H), lambda i: (0, 0)),
            pl.BlockSpec((1, H), lambda i: (0, 0)),
            pl.BlockSpec((H, NF), lambda i: (0, 0)),
        ],
        out_specs=[
            pl.BlockSpec((_BN, H), lambda i: (i, 0)),
            pl.BlockSpec((_BN, NF), lambda i: (i, 0)),
        ],
        out_shape=[
            jax.ShapeDtypeStruct((N, H), jnp.float32),
            jax.ShapeDtypeStruct((N, NF), jnp.float32),
        ],
        compiler_params=pltpu.CompilerParams(
            dimension_semantics=("arbitrary",),
        ),
    )(h, ph, pc, b2, a_t, v, l1b, c1w_t)


def _node_post_pool_body(a0_ref, a1_ref, h1_ref, b_ref, c2w_ref, c2b_ref,
                         ilw_ref, ilb_ref, h2_ref, ph_ref, pc_ref):
    agg = a0_ref[...] + a1_ref[...]
    t = _ssp(jnp.dot(agg, c2w_ref[...], preferred_element_type=jnp.float32)
             + c2b_ref[...])
    h2 = (h1_ref[...]
          + jnp.dot(t, ilw_ref[...], preferred_element_type=jnp.float32)
          + ilb_ref[...])
    h2_ref[...] = h2
    onehot = (b_ref[...] ==
              lax.broadcasted_iota(jnp.int32, (_BN, G), 1)).astype(jnp.float32)
    ph = lax.dot_general(onehot, h2, (((0,), (0,)), ((), ())),
                         preferred_element_type=jnp.float32)
    pc = jnp.sum(onehot, axis=0)[:, None]
    i = pl.program_id(0)

    @pl.when(i == 0)
    def _():
        ph_ref[...] = ph
        pc_ref[...] = pc

    @pl.when(i != 0)
    def _():
        ph_ref[...] += ph
        pc_ref[...] += pc


def _node_post_pool(agg2, h1, b2, c2w_t, c2b, ilw_t, ilb):
    grid = (N // _BN,)
    nb = N // _BN
    return pl.pallas_call(
        _node_post_pool_body,
        grid=grid,
        in_specs=[
            pl.BlockSpec((_BN, NF), lambda i: (i, 0)),
            pl.BlockSpec((_BN, NF), lambda i, nb=nb: (i + nb, 0)),
            pl.BlockSpec((_BN, H), lambda i: (i, 0)),
            pl.BlockSpec((_BN, 1), lambda i: (i, 0)),
            pl.BlockSpec((NF, H), lambda i: (0, 0)),
            pl.BlockSpec((1, H), lambda i: (0, 0)),
            pl.BlockSpec((H, H), lambda i: (0, 0)),
            pl.BlockSpec((1, H), lambda i: (0, 0)),
        ],
        out_specs=[
            pl.BlockSpec((_BN, H), lambda i: (i, 0)),
            pl.BlockSpec((G, H), lambda i: (0, 0)),
            pl.BlockSpec((G, 1), lambda i: (0, 0)),
        ],
        out_shape=[
            jax.ShapeDtypeStruct((N, H), jnp.float32),
            jax.ShapeDtypeStruct((G, H), jnp.float32),
            jax.ShapeDtypeStruct((G, 1), jnp.float32),
        ],
        compiler_params=pltpu.CompilerParams(
            dimension_semantics=("arbitrary",),
        ),
    )(agg2, agg2, h1, b2, c2w_t, c2b, ilw_t, ilb)


def _node_post_final_body(a0_ref, a1_ref, h1_ref, c2w_ref, c2b_ref,
                          ilw_ref, ilb_ref, ow_ref, ob_ref, out_ref):
    agg = a0_ref[...] + a1_ref[...]
    t = _ssp(jnp.dot(agg, c2w_ref[...], preferred_element_type=jnp.float32)
             + c2b_ref[...])
    h2 = (h1_ref[...]
          + jnp.dot(t, ilw_ref[...], preferred_element_type=jnp.float32)
          + ilb_ref[...])
    out_ref[...] = jnp.maximum(
        jnp.dot(h2, ow_ref[...], preferred_element_type=jnp.float32)
        + ob_ref[...], 0.0)


def _node_post_final(agg2, h1, c2w_t, c2b, ilw_t, ilb, ow_t, ob):
    grid = (N // _BN,)
    nb = N // _BN
    return pl.pallas_call(
        _node_post_final_body,
        grid=grid,
        in_specs=[
            pl.BlockSpec((_BN, NF), lambda i: (i, 0)),
            pl.BlockSpec((_BN, NF), lambda i, nb=nb: (i + nb, 0)),
            pl.BlockSpec((_BN, H), lambda i: (i, 0)),
            pl.BlockSpec((NF, H), lambda i: (0, 0)),
            pl.BlockSpec((1, H), lambda i: (0, 0)),
            pl.BlockSpec((H, H), lambda i: (0, 0)),
            pl.BlockSpec((1, H), lambda i: (0, 0)),
            pl.BlockSpec((H, NF), lambda i: (0, 0)),
            pl.BlockSpec((1, NF), lambda i: (0, 0)),
        ],
        out_specs=pl.BlockSpec((_BN, NF), lambda i: (i, 0)),
        out_shape=jax.ShapeDtypeStruct((N, NF), jnp.float32),
        compiler_params=pltpu.CompilerParams(
            dimension_semantics=("arbitrary",),
        ),
    )(agg2, agg2, h1, c2w_t, c2b, ilw_t, ilb, ow_t, ob)


_NC = 2
_NSUB = 16
_NW = _NC * _NSUB
_CH = 80
_NCHUNK = (E // _NW) // _CH
_NSLOT = 4
_RPW = 624
_RREM = N - _NSUB * _RPW
assert _NCHUNK % _NSLOT != 0


def _edge_agg_body(xl_hbm, we_hbm, idx_hbm, out_hbm,
                   idx_v, we_v, rows_v, acc_sh,
                   sidx, swe, sg, ss):
    cid = lax.axis_index("c")
    sid = lax.axis_index("s")
    wid = sid * _NC + cid
    c0 = wid * _NCHUNK

    def idx_cp(ci, s):
        return pltpu.make_async_copy(idx_hbm.at[c0 + ci], idx_v.at[s], sidx[s])

    def we_cp(ci, s):
        return pltpu.make_async_copy(
            we_hbm.at[pl.ds((c0 + ci) * _CH, _CH)], we_v.at[s], swe[s])

    def gat_cp(s):
        return pltpu.make_async_copy(xl_hbm.at[idx_v.at[s, 0]],
                                     rows_v.at[s], sg[s])

    def sca_cp(s):
        return pltpu.make_async_copy(rows_v.at[s],
                                     acc_sh.at[idx_v.at[s, 1]], ss[s])

    def _zrow(i, _):
        for j in range(NF // 16):
            rows_v[0, i, pl.ds(j * 16, 16)] = jnp.zeros((16,), jnp.float32)
        return 0
    lax.fori_loop(0, _CH, _zrow, 0)
    base_r = sid * _RPW
    done = 0
    while done < _RPW:
        n = min(_CH, _RPW - done)
        pltpu.sync_copy(rows_v.at[0, pl.ds(0, n)],
                        acc_sh.at[pl.ds(base_r + done, n)])
        done += n

    @pl.when(sid == _NSUB - 1)
    def _():
        pltpu.sync_copy(rows_v.at[0, pl.ds(0, _RREM)],
                        acc_sh.at[pl.ds(_NSUB * _RPW, _RREM)])
    plsc.subcore_barrier()

    idx_cp(0, 0).start()
    we_cp(0, 0).start()
    idx_cp(1, 1).start()
    we_cp(1, 1).start()
    idx_cp(0, 0).wait()
    gat_cp(0).start()

    def _step(ci, s):
        s1 = (s + 1) % _NSLOT
        s2 = (s + 2) % _NSLOT

        @pl.when(jnp.logical_and(ci >= 2, ci <= _NCHUNK + 1))
        def _():
            sca_cp(s2).wait()

        @pl.when(ci + 2 < _NCHUNK)
        def _():
            idx_cp(ci + 2, s2).start()
            we_cp(ci + 2, s2).start()

        @pl.when(ci + 1 < _NCHUNK)
        def _():
            idx_cp(ci + 1, s1).wait()
            gat_cp(s1).start()

        @pl.when(ci < _NCHUNK)
        def _():
            gat_cp(s).wait()
            we_cp(ci, s).wait()

            def _mul(i, _):
                for r in range(4):
                    for j in range(NF // 16):
                        sl = pl.ds(j * 16, 16)
                        rows_v[s, 4 * i + r, sl] = (rows_v[s, 4 * i + r, sl]
                                                    * we_v[s, 4 * i + r, sl])
                return 0
            lax.fori_loop(0, _CH // 4, _mul, 0)
            sca_cp(s).start(add=True)

    def _quad(i, _):
        for k in range(_NSLOT):
            _step(_NSLOT * i + k, k)
        return 0
    lax.fori_loop(0, _NCHUNK // _NSLOT + 1, _quad, 0)

    plsc.subcore_barrier()
    pltpu.sync_copy(acc_sh.at[pl.ds(base_r, _RPW)],
                    out_hbm.at[pl.ds(cid * N + base_r, _RPW)])

    @pl.when(sid == _NSUB - 1)
    def _():
        pltpu.sync_copy(acc_sh.at[pl.ds(_NSUB * _RPW, _RREM)],
                        out_hbm.at[pl.ds(cid * N + _NSUB * _RPW, _RREM)])


def _edge_agg(xl, we, idx_pairs):
    mesh = plsc.VectorSubcoreMesh(core_axis_name="c", subcore_axis_name="s")
    f = pl.kernel(
        _edge_agg_body,
        out_type=jax.ShapeDtypeStruct((_NC * N, NF), jnp.float32),
        mesh=mesh,
        scratch_types=[
            pltpu.VMEM((_NSLOT, 2, _CH), jnp.int32),
            pltpu.VMEM((_NSLOT, _CH, NF), jnp.float32),
            pltpu.VMEM((_NSLOT, _CH, NF), jnp.float32),
            pltpu.VMEM_SHARED((N, NF), jnp.float32),
            [pltpu.SemaphoreType.DMA] * _NSLOT,
            [pltpu.SemaphoreType.DMA] * _NSLOT,
            [pltpu.SemaphoreType.DMA] * _NSLOT,
            [pltpu.SemaphoreType.DMA] * _NSLOT,
        ],
        compiler_params=pltpu.CompilerParams(use_tc_tiling_on_sc=False),
    )
    return f(xl, we, idx_pairs)


def kernel(h, edge_index, edge_weight, edge_attr, state_attr, batch,
           lin1_w_0, lin1_b_0, mlp1_w_0, mlp1_b_0, mlp2_w_0, mlp2_b_0,
           cf1_w_0, cf2_w_0, cf2_b_0, il_w_0, il_b_0,
           lin1_w_1, lin1_b_1, mlp1_w_1, mlp1_b_1, mlp2_w_1, mlp2_b_1,
           cf1_w_1, cf2_w_1, cf2_b_1, il_w_1, il_b_1, out_w, out_b):
    b2 = batch[:, None]
    ew_mat = edge_weight.reshape(E // 128, 128)
    idx_pairs = edge_index.reshape(2, E // _CH, _CH).transpose(1, 0, 2)

    zblk = jnp.zeros((NF, NF), jnp.float32)
    m1cat = jnp.concatenate([mlp1_w_0.T, mlp1_w_1.T], axis=1)
    m1bcat = jnp.concatenate([mlp1_b_0, mlp1_b_1])[None, :]
    m2blk = jnp.block([[mlp2_w_0.T, zblk], [zblk, mlp2_w_1.T]])
    m2bcat = jnp.concatenate([mlp2_b_0, mlp2_b_1])[None, :]
    we0, we1 = _edge_filter(edge_attr, ew_mat, m1cat, m1bcat, m2blk, m2bcat)

    ph0 = jnp.pad(state_attr, ((0, 0), (0, H - NS)))
    pc0 = jnp.ones((G, 1), jnp.float32)

    h1, xl = _node_pre(h, ph0, pc0, b2,
                       lin1_w_0[:, NS:].T, lin1_w_0[:, :NS].T,
                       lin1_b_0[None, :], cf1_w_0.T)
    agg2 = _edge_agg(xl, we0, idx_pairs)
    h2, ph, pc = _node_post_pool(agg2, h1, b2, cf2_w_0.T, cf2_b_0[None, :],
                                 il_w_0.T, il_b_0[None, :])

    h3, xl1 = _node_pre(h2, ph, pc, b2,
                        lin1_w_1[:, NS:].T, lin1_w_1[:, :NS].T,
                        lin1_b_1[None, :], cf1_w_1.T)
    agg2b = _edge_agg(xl1, we1, idx_pairs)
    return _node_post_final(agg2b, h3, cf2_w_1.T, cf2_b_1[None, :],
                            il_w_1.T, il_b_1[None, :], out_w.T,
                            out_b[None, :])


# speedup vs baseline: 4.4409x; 1.0228x over previous
import functools
import math

import jax
import jax.numpy as jnp
from jax import lax
from jax.experimental import pallas as pl
from jax.experimental.pallas import tpu as pltpu
from jax.experimental.pallas import tpu_sc as plsc

N = 10000
E = 320000
H = 128
NG = 16
NF = 64
G = 64
NS = 1
CUTOFF = 10.0
LOG2 = math.log(2.0)


_BE = 3072
_BN = 1000


def _ssp(x):
    return jnp.maximum(x, 0.0) + jnp.log(1.0 + jnp.exp(-jnp.abs(x))) - LOG2


def _cos_poly(t):
    u = t * t
    return 1.0 + u * (-0.5 + u * (1.0 / 24.0 + u * (-1.0 / 720.0
                                                    + u * (1.0 / 40320.0))))


def _edge_filter_body(ea_ref, ew_ref, m1w_ref, m1b_ref, m2w_ref, m2b_ref,
                      wec_ref):
    z = _ssp(jnp.dot(ea_ref[...], m1w_ref[...],
                     preferred_element_type=jnp.float32) + m1b_ref[...])
    z2 = (jnp.dot(z, m2w_ref[...], preferred_element_type=jnp.float32)
          + m2b_ref[...])
    c = 0.5 * (_cos_poly(ew_ref[...] * (math.pi / CUTOFF)) + 1.0)
    grow = (pl.program_id(0) * (_BE // 128)
            + lax.broadcasted_iota(jnp.int32, (_BE // 128, 128), 0))
    c = jnp.where(grow < E // 128, c, 0.0)
    rows = lax.broadcasted_iota(jnp.int32, (_BE, 128), 0)
    asel = (rows // 128 ==
            lax.broadcasted_iota(jnp.int32, (_BE, 128), 1)).astype(jnp.float32)
    msel = (rows % 128 ==
            lax.broadcasted_iota(jnp.int32, (_BE, 128), 1)).astype(jnp.float32)
    cpad = jnp.concatenate(
        [c, jnp.zeros((128 - _BE // 128, 128), jnp.float32)], axis=0)
    c_sel = jnp.dot(asel, cpad, preferred_element_type=jnp.float32) * msel
    c_rep = jnp.dot(c_sel, jnp.ones((128, 2 * NF), jnp.float32),
                    preferred_element_type=jnp.float32)
    wec_ref[...] = z2 * c_rep


def _edge_filter(ea, ew_mat, m1w, m1b, m2w, m2b):
    grid = (pl.cdiv(E, _BE),)
    return pl.pallas_call(
        _edge_filter_body,
        grid=grid,
        in_specs=[
            pl.BlockSpec((_BE, NG), lambda i: (i, 0)),
            pl.BlockSpec((_BE // 128, 128), lambda i: (i, 0)),
            pl.BlockSpec((NG, 2 * NF), lambda i: (0, 0)),
            pl.BlockSpec((1, 2 * NF), lambda i: (0, 0)),
            pl.BlockSpec((2 * NF, 2 * NF), lambda i: (0, 0)),
            pl.BlockSpec((1, 2 * NF), lambda i: (0, 0)),
        ],
        out_specs=pl.BlockSpec((_BE, 2 * NF), lambda i: (i, 0)),
        out_shape=jax.ShapeDtypeStruct((E, 2 * NF), jnp.float32),
        compiler_params=pltpu.CompilerParams(
            dimension_semantics=("arbitrary",),
        ),
    )(ea, ew_mat, m1w, m1b, m2w, m2b)


def _node_pre_body(h_ref, ph_ref, pc_ref, b_ref, a_ref, v_ref, l1b_ref,
                   c1w_ref, h1_ref, xl_ref):
    sa = (jnp.sum(ph_ref[...], axis=1, keepdims=True)
          / jnp.maximum(pc_ref[...], 1.0))
    onehot = (b_ref[...] ==
              lax.broadcasted_iota(jnp.int32, (_BN, G), 1)).astype(jnp.float32)
    s = jnp.dot(onehot, sa, preferred_element_type=jnp.float32)
    h1 = (jnp.dot(h_ref[...], a_ref[...], preferred_element_type=jnp.float32)
          + s * v_ref[...] + l1b_ref[...])
    h1_ref[...] = h1
    xl_ref[...] = jnp.dot(h1, c1w_ref[...], preferred_element_type=jnp.float32)


def _node_pre(h, ph, pc, b2, a_t, v, l1b, c1w_t):
    grid = (N // _BN,)
    return pl.pallas_call(
        _node_pre_body,
        grid=grid,
        in_specs=[
            pl.BlockSpec((_BN, H), lambda i: (i, 0)),
            pl.BlockSpec((G, H), lambda i: (0, 0)),
            pl.BlockSpec((G, 1), lambda i: (0, 0)),
            pl.BlockSpec((_BN, 1), lambda i: (i, 0)),
            pl.BlockSpec((H, H), lambda i: (0, 0)),
            pl.BlockSpec((1, H), lambda i: (0, 0)),
            pl.BlockSpec((1, H), lambda i: (0, 0)),
            pl.BlockSpec((H, NF), lambda i: (0, 0)),
        ],
        out_specs=[
            pl.BlockSpec((_BN, H), lambda i: (i, 0)),
            pl.BlockSpec((_BN, NF), lambda i: (i, 0)),
        ],
        out_shape=[
            jax.ShapeDtypeStruct((N, H), jnp.float32),
            jax.ShapeDtypeStruct((N, NF), jnp.float32),
        ],
        compiler_params=pltpu.CompilerParams(
            dimension_semantics=("arbitrary",),
        ),
    )(h, ph, pc, b2, a_t, v, l1b, c1w_t)


def _node_post_pool_body(a0_ref, a1_ref, h1_ref, b_ref, c2w_ref, c2b_ref,
                         ilw_ref, ilb_ref, h2_ref, ph_ref, pc_ref):
    agg = a0_ref[...] + a1_ref[...]
    t = _ssp(jnp.dot(agg, c2w_ref[...], preferred_element_type=jnp.float32)
             + c2b_ref[...])
    h2 = (h1_ref[...]
          + jnp.dot(t, ilw_ref[...], preferred_element_type=jnp.float32)
          + ilb_ref[...])
    h2_ref[...] = h2
    onehot = (b_ref[...] ==
              lax.broadcasted_iota(jnp.int32, (_BN, G), 1)).astype(jnp.float32)
    ph = lax.dot_general(onehot, h2, (((0,), (0,)), ((), ())),
                         preferred_element_type=jnp.float32)
    pc = jnp.sum(onehot, axis=0)[:, None]
    i = pl.program_id(0)

    @pl.when(i == 0)
    def _():
        ph_ref[...] = ph
        pc_ref[...] = pc

    @pl.when(i != 0)
    def _():
        ph_ref[...] += ph
        pc_ref[...] += pc


def _node_post_pool(agg2, h1, b2, c2w_t, c2b, ilw_t, ilb):
    grid = (N // _BN,)
    nb = N // _BN
    return pl.pallas_call(
        _node_post_pool_body,
        grid=grid,
        in_specs=[
            pl.BlockSpec((_BN, NF), lambda i: (i, 0)),
            pl.BlockSpec((_BN, NF), lambda i, nb=nb: (i + nb, 0)),
            pl.BlockSpec((_BN, H), lambda i: (i, 0)),
            pl.BlockSpec((_BN, 1), lambda i: (i, 0)),
            pl.BlockSpec((NF, H), lambda i: (0, 0)),
            pl.BlockSpec((1, H), lambda i: (0, 0)),
            pl.BlockSpec((H, H), lambda i: (0, 0)),
            pl.BlockSpec((1, H), lambda i: (0, 0)),
        ],
        out_specs=[
            pl.BlockSpec((_BN, H), lambda i: (i, 0)),
            pl.BlockSpec((G, H), lambda i: (0, 0)),
            pl.BlockSpec((G, 1), lambda i: (0, 0)),
        ],
        out_shape=[
            jax.ShapeDtypeStruct((N, H), jnp.float32),
            jax.ShapeDtypeStruct((G, H), jnp.float32),
            jax.ShapeDtypeStruct((G, 1), jnp.float32),
        ],
        compiler_params=pltpu.CompilerParams(
            dimension_semantics=("arbitrary",),
        ),
    )(agg2, agg2, h1, b2, c2w_t, c2b, ilw_t, ilb)


def _node_post_final_body(a0_ref, a1_ref, h1_ref, c2w_ref, c2b_ref,
                          ilw_ref, ilb_ref, ow_ref, ob_ref, out_ref):
    agg = a0_ref[...] + a1_ref[...]
    t = _ssp(jnp.dot(agg, c2w_ref[...], preferred_element_type=jnp.float32)
             + c2b_ref[...])
    h2 = (h1_ref[...]
          + jnp.dot(t, ilw_ref[...], preferred_element_type=jnp.float32)
          + ilb_ref[...])
    out_ref[...] = jnp.maximum(
        jnp.dot(h2, ow_ref[...], preferred_element_type=jnp.float32)
        + ob_ref[...], 0.0)


def _node_post_final(agg2, h1, c2w_t, c2b, ilw_t, ilb, ow_t, ob):
    grid = (N // _BN,)
    nb = N // _BN
    return pl.pallas_call(
        _node_post_final_body,
        grid=grid,
        in_specs=[
            pl.BlockSpec((_BN, NF), lambda i: (i, 0)),
            pl.BlockSpec((_BN, NF), lambda i, nb=nb: (i + nb, 0)),
            pl.BlockSpec((_BN, H), lambda i: (i, 0)),
            pl.BlockSpec((NF, H), lambda i: (0, 0)),
            pl.BlockSpec((1, H), lambda i: (0, 0)),
            pl.BlockSpec((H, H), lambda i: (0, 0)),
            pl.BlockSpec((1, H), lambda i: (0, 0)),
            pl.BlockSpec((H, NF), lambda i: (0, 0)),
            pl.BlockSpec((1, NF), lambda i: (0, 0)),
        ],
        out_specs=pl.BlockSpec((_BN, NF), lambda i: (i, 0)),
        out_shape=jax.ShapeDtypeStruct((N, NF), jnp.float32),
        compiler_params=pltpu.CompilerParams(
            dimension_semantics=("arbitrary",),
        ),
    )(agg2, agg2, h1, c2w_t, c2b, ilw_t, ilb, ow_t, ob)


_NC = 2
_NSUB = 16
_NW = _NC * _NSUB
_CH = 80
_NCHUNK = (E // _NW) // _CH
_NSLOT = 4
_TRIPS = _NCHUNK // _NSLOT + 1
_RPW = 624
_RREM = N - _NSUB * _RPW
assert _NCHUNK % _NSLOT != 0


def _make_edge_agg_body(col0):
    def _edge_agg_body(xl_hbm, we_hbm, src_hbm, dst_hbm, out_hbm,
                       idx_v, we_v, rows_v, acc_sh,
                       sidx, swe, sg, ss):
        cid = lax.axis_index("c")
        sid = lax.axis_index("s")
        wid = sid * _NC + cid
        c0 = wid * _NCHUNK

        def idx_cp(ci, s, d):
            hbm = dst_hbm if d else src_hbm
            return pltpu.make_async_copy(hbm.at[c0 + ci], idx_v.at[s, d],
                                         sidx[s])

        def we_cp(ci, s):
            return pltpu.make_async_copy(
                we_hbm.at[pl.ds((c0 + ci) * _CH, _CH)], we_v.at[s], swe[s])

        def gat_cp(s):
            return pltpu.make_async_copy(xl_hbm.at[idx_v.at[s, 0]],
                                         rows_v.at[s], sg[s])

        def sca_cp(s):
            return pltpu.make_async_copy(rows_v.at[s],
                                         acc_sh.at[idx_v.at[s, 1]], ss[s])

        def _zrow(i, _):
            for j in range(NF // 16):
                rows_v[0, i, pl.ds(j * 16, 16)] = jnp.zeros((16,), jnp.float32)
            return 0
        lax.fori_loop(0, _CH, _zrow, 0)
        base_r = sid * _RPW
        done = 0
        while done < _RPW:
            n = min(_CH, _RPW - done)
            pltpu.sync_copy(rows_v.at[0, pl.ds(0, n)],
                            acc_sh.at[pl.ds(base_r + done, n)])
            done += n

        @pl.when(sid == _NSUB - 1)
        def _():
            pltpu.sync_copy(rows_v.at[0, pl.ds(0, _RREM)],
                            acc_sh.at[pl.ds(_NSUB * _RPW, _RREM)])
        plsc.subcore_barrier()

        idx_cp(0, 0, 0).start()
        idx_cp(0, 0, 1).start()
        we_cp(0, 0).start()
        idx_cp(1, 1, 0).start()
        idx_cp(1, 1, 1).start()
        we_cp(1, 1).start()
        idx_cp(0, 0, 0).wait()
        idx_cp(0, 0, 1).wait()
        gat_cp(0).start()

        def _step(ci, s):
            s1 = (s + 1) % _NSLOT
            s2 = (s + 2) % _NSLOT

            @pl.when(jnp.logical_and(ci >= 2, ci <= _NCHUNK + 1))
            def _():
                sca_cp(s2).wait()

            @pl.when(ci + 2 < _NCHUNK)
            def _():
                idx_cp(ci + 2, s2, 0).start()
                idx_cp(ci + 2, s2, 1).start()
                we_cp(ci + 2, s2).start()

            @pl.when(ci + 1 < _NCHUNK)
            def _():
                idx_cp(ci + 1, s1, 0).wait()
                idx_cp(ci + 1, s1, 1).wait()
                gat_cp(s1).start()

            @pl.when(ci < _NCHUNK)
            def _():
                gat_cp(s).wait()
                we_cp(ci, s).wait()

                def _mul(i, _):
                    for r in range(4):
                        for j in range(NF // 16):
                            sl = pl.ds(j * 16, 16)
                            sw = pl.ds(col0 + j * 16, 16)
                            rows_v[s, 4 * i + r, sl] = (
                                rows_v[s, 4 * i + r, sl]
                                * we_v[s, 4 * i + r, sw])
                    return 0
                lax.fori_loop(0, _CH // 4, _mul, 0)
                sca_cp(s).start(add=True)

        def _quad(i, _):
            for k in range(_NSLOT):
                _step(_NSLOT * i + k, k)
            return 0
        lax.fori_loop(0, _TRIPS, _quad, 0)

        plsc.subcore_barrier()
        pltpu.sync_copy(acc_sh.at[pl.ds(base_r, _RPW)],
                        out_hbm.at[pl.ds(cid * N + base_r, _RPW)])

        @pl.when(sid == _NSUB - 1)
        def _():
            pltpu.sync_copy(acc_sh.at[pl.ds(_NSUB * _RPW, _RREM)],
                            out_hbm.at[pl.ds(cid * N + _NSUB * _RPW, _RREM)])

    return _edge_agg_body


@functools.cache
def _edge_agg_fn(col0):
    mesh = plsc.VectorSubcoreMesh(core_axis_name="c", subcore_axis_name="s")
    return pl.kernel(
        _make_edge_agg_body(col0),
        out_type=jax.ShapeDtypeStruct((_NC * N, NF), jnp.float32),
        mesh=mesh,
        scratch_types=[
            pltpu.VMEM((_NSLOT, 2, _CH), jnp.int32),
            pltpu.VMEM((_NSLOT, _CH, 2 * NF), jnp.float32),
            pltpu.VMEM((_NSLOT, _CH, NF), jnp.float32),
            pltpu.VMEM_SHARED((N, NF), jnp.float32),
            [pltpu.SemaphoreType.DMA] * _NSLOT,
            [pltpu.SemaphoreType.DMA] * _NSLOT,
            [pltpu.SemaphoreType.DMA] * _NSLOT,
            [pltpu.SemaphoreType.DMA] * _NSLOT,
        ],
        compiler_params=pltpu.CompilerParams(use_tc_tiling_on_sc=False),
    )


def _edge_agg(xl, we, src2, dst2, it):
    return _edge_agg_fn(it * NF)(xl, we, src2, dst2)


def kernel(h, edge_index, edge_weight, edge_attr, state_attr, batch,
           lin1_w_0, lin1_b_0, mlp1_w_0, mlp1_b_0, mlp2_w_0, mlp2_b_0,
           cf1_w_0, cf2_w_0, cf2_b_0, il_w_0, il_b_0,
           lin1_w_1, lin1_b_1, mlp1_w_1, mlp1_b_1, mlp2_w_1, mlp2_b_1,
           cf1_w_1, cf2_w_1, cf2_b_1, il_w_1, il_b_1, out_w, out_b):
    b2 = batch[:, None]
    ew_mat = edge_weight.reshape(E // 128, 128)
    src2 = edge_index[0].reshape(E // _CH, _CH)
    dst2 = edge_index[1].reshape(E // _CH, _CH)

    zblk = jnp.zeros((NF, NF), jnp.float32)
    m1cat = jnp.concatenate([mlp1_w_0.T, mlp1_w_1.T], axis=1)
    m1bcat = jnp.concatenate([mlp1_b_0, mlp1_b_1])[None, :]
    m2blk = jnp.block([[mlp2_w_0.T, zblk], [zblk, mlp2_w_1.T]])
    m2bcat = jnp.concatenate([mlp2_b_0, mlp2_b_1])[None, :]
    wec = _edge_filter(edge_attr, ew_mat, m1cat, m1bcat, m2blk, m2bcat)

    ph0 = jnp.pad(state_attr, ((0, 0), (0, H - NS)))
    pc0 = jnp.ones((G, 1), jnp.float32)

    h1, xl = _node_pre(h, ph0, pc0, b2,
                       lin1_w_0[:, NS:].T, lin1_w_0[:, :NS].T,
                       lin1_b_0[None, :], cf1_w_0.T)
    agg2 = _edge_agg(xl, wec, src2, dst2, 0)
    h2, ph, pc = _node_post_pool(agg2, h1, b2, cf2_w_0.T, cf2_b_0[None, :],
                                 il_w_0.T, il_b_0[None, :])

    h3, xl1 = _node_pre(h2, ph, pc, b2,
                        lin1_w_1[:, NS:].T, lin1_w_1[:, :NS].T,
                        lin1_b_1[None, :], cf1_w_1.T)
    agg2b = _edge_agg(xl1, wec, src2, dst2, 1)
    return _node_post_final(agg2b, h3, cf2_w_1.T, cf2_b_1[None, :],
                            il_w_1.T, il_b_1[None, :], out_w.T,
                            out_b[None, :])


# speedup vs baseline: 5.9198x; 1.3330x over previous
import functools
import math

import jax
import jax.numpy as jnp
from jax import lax
from jax.experimental import pallas as pl
from jax.experimental.pallas import tpu as pltpu
from jax.experimental.pallas import tpu_sc as plsc

N = 10000
E = 320000
H = 128
NG = 16
NF = 64
G = 64
NS = 1
CUTOFF = 10.0
LOG2 = math.log(2.0)


_BE = 3072
_BN = 1000


def _ssp(x):
    return jnp.maximum(x, 0.0) + jnp.log(1.0 + jnp.exp(-jnp.abs(x))) - LOG2


def _cos_poly(t):
    u = t * t
    return 1.0 + u * (-0.5 + u * (1.0 / 24.0 + u * (-1.0 / 720.0
                                                    + u * (1.0 / 40320.0))))


def _edge_filter_body(ea_ref, ew_ref, m1w_ref, m1b_ref, m2w_ref, m2b_ref,
                      wec_ref):
    z = _ssp(jnp.dot(ea_ref[...], m1w_ref[...],
                     preferred_element_type=jnp.float32) + m1b_ref[...])
    z2 = (jnp.dot(z, m2w_ref[...], preferred_element_type=jnp.float32)
          + m2b_ref[...])
    c = 0.5 * (_cos_poly(ew_ref[...] * (math.pi / CUTOFF)) + 1.0)
    grow = (pl.program_id(0) * (_BE // 128)
            + lax.broadcasted_iota(jnp.int32, (_BE // 128, 128), 0))
    c = jnp.where(grow < E // 128, c, 0.0)
    rows = lax.broadcasted_iota(jnp.int32, (_BE, 128), 0)
    asel = (rows // 128 ==
            lax.broadcasted_iota(jnp.int32, (_BE, 128), 1)).astype(jnp.float32)
    msel = (rows % 128 ==
            lax.broadcasted_iota(jnp.int32, (_BE, 128), 1)).astype(jnp.float32)
    cpad = jnp.concatenate(
        [c, jnp.zeros((128 - _BE // 128, 128), jnp.float32)], axis=0)
    c_sel = jnp.dot(asel, cpad, preferred_element_type=jnp.float32) * msel
    c_rep = jnp.dot(c_sel, jnp.ones((128, 2 * NF), jnp.float32),
                    preferred_element_type=jnp.float32)
    wec_ref[...] = z2 * c_rep


def _edge_filter(ea, ew_mat, m1w, m1b, m2w, m2b):
    grid = (pl.cdiv(E, _BE),)
    return pl.pallas_call(
        _edge_filter_body,
        grid=grid,
        in_specs=[
            pl.BlockSpec((_BE, NG), lambda i: (i, 0)),
            pl.BlockSpec((_BE // 128, 128), lambda i: (i, 0)),
            pl.BlockSpec((NG, 2 * NF), lambda i: (0, 0)),
            pl.BlockSpec((1, 2 * NF), lambda i: (0, 0)),
            pl.BlockSpec((2 * NF, 2 * NF), lambda i: (0, 0)),
            pl.BlockSpec((1, 2 * NF), lambda i: (0, 0)),
        ],
        out_specs=pl.BlockSpec((_BE, 2 * NF), lambda i: (i, 0)),
        out_shape=jax.ShapeDtypeStruct((E, 2 * NF), jnp.float32),
        compiler_params=pltpu.CompilerParams(
            dimension_semantics=("arbitrary",),
        ),
    )(ea, ew_mat, m1w, m1b, m2w, m2b)


def _node_pre_body(h_ref, ph_ref, pc_ref, b_ref, a_ref, v_ref, l1b_ref,
                   c1w_ref, h1_ref, xl_ref):
    sa = (jnp.sum(ph_ref[...], axis=1, keepdims=True)
          / jnp.maximum(pc_ref[...], 1.0))
    onehot = (b_ref[...] ==
              lax.broadcasted_iota(jnp.int32, (_BN, G), 1)).astype(jnp.float32)
    s = jnp.dot(onehot, sa, preferred_element_type=jnp.float32)
    h1 = (jnp.dot(h_ref[...], a_ref[...], preferred_element_type=jnp.float32)
          + s * v_ref[...] + l1b_ref[...])
    h1_ref[...] = h1
    xl_ref[...] = jnp.dot(h1, c1w_ref[...], preferred_element_type=jnp.float32)


def _node_pre(h, ph, pc, b2, a_t, v, l1b, c1w_t):
    grid = (N // _BN,)
    return pl.pallas_call(
        _node_pre_body,
        grid=grid,
        in_specs=[
            pl.BlockSpec((_BN, H), lambda i: (i, 0)),
            pl.BlockSpec((G, H), lambda i: (0, 0)),
            pl.BlockSpec((G, 1), lambda i: (0, 0)),
            pl.BlockSpec((_BN, 1), lambda i: (i, 0)),
            pl.BlockSpec((H, H), lambda i: (0, 0)),
            pl.BlockSpec((1, H), lambda i: (0, 0)),
            pl.BlockSpec((1, H), lambda i: (0, 0)),
            pl.BlockSpec((H, NF), lambda i: (0, 0)),
        ],
        out_specs=[
            pl.BlockSpec((_BN, H), lambda i: (i, 0)),
            pl.BlockSpec((_BN, NF), lambda i: (i, 0)),
        ],
        out_shape=[
            jax.ShapeDtypeStruct((N, H), jnp.float32),
            jax.ShapeDtypeStruct((N, NF), jnp.float32),
        ],
        compiler_params=pltpu.CompilerParams(
            dimension_semantics=("arbitrary",),
        ),
    )(h, ph, pc, b2, a_t, v, l1b, c1w_t)


def _node_post_pool_body(a0_ref, a1_ref, h1_ref, b_ref, c2w_ref, c2b_ref,
                         ilw_ref, ilb_ref, h2_ref, ph_ref, pc_ref):
    agg = a0_ref[...] + a1_ref[...]
    t = _ssp(jnp.dot(agg, c2w_ref[...], preferred_element_type=jnp.float32)
             + c2b_ref[...])
    h2 = (h1_ref[...]
          + jnp.dot(t, ilw_ref[...], preferred_element_type=jnp.float32)
          + ilb_ref[...])
    h2_ref[...] = h2
    onehot = (b_ref[...] ==
              lax.broadcasted_iota(jnp.int32, (_BN, G), 1)).astype(jnp.float32)
    ph = lax.dot_general(onehot, h2, (((0,), (0,)), ((), ())),
                         preferred_element_type=jnp.float32)
    pc = jnp.sum(onehot, axis=0)[:, None]
    i = pl.program_id(0)

    @pl.when(i == 0)
    def _():
        ph_ref[...] = ph
        pc_ref[...] = pc

    @pl.when(i != 0)
    def _():
        ph_ref[...] += ph
        pc_ref[...] += pc


def _node_post_pool(agg2, h1, b2, c2w_t, c2b, ilw_t, ilb):
    grid = (N // _BN,)
    nb = N // _BN
    return pl.pallas_call(
        _node_post_pool_body,
        grid=grid,
        in_specs=[
            pl.BlockSpec((_BN, NF), lambda i: (i, 0)),
            pl.BlockSpec((_BN, NF), lambda i, nb=nb: (i + nb, 0)),
            pl.BlockSpec((_BN, H), lambda i: (i, 0)),
            pl.BlockSpec((_BN, 1), lambda i: (i, 0)),
            pl.BlockSpec((NF, H), lambda i: (0, 0)),
            pl.BlockSpec((1, H), lambda i: (0, 0)),
            pl.BlockSpec((H, H), lambda i: (0, 0)),
            pl.BlockSpec((1, H), lambda i: (0, 0)),
        ],
        out_specs=[
            pl.BlockSpec((_BN, H), lambda i: (i, 0)),
            pl.BlockSpec((G, H), lambda i: (0, 0)),
            pl.BlockSpec((G, 1), lambda i: (0, 0)),
        ],
        out_shape=[
            jax.ShapeDtypeStruct((N, H), jnp.float32),
            jax.ShapeDtypeStruct((G, H), jnp.float32),
            jax.ShapeDtypeStruct((G, 1), jnp.float32),
        ],
        compiler_params=pltpu.CompilerParams(
            dimension_semantics=("arbitrary",),
        ),
    )(agg2, agg2, h1, b2, c2w_t, c2b, ilw_t, ilb)


def _node_post_final_body(a0_ref, a1_ref, h1_ref, c2w_ref, c2b_ref,
                          ilw_ref, ilb_ref, ow_ref, ob_ref, out_ref):
    agg = a0_ref[...] + a1_ref[...]
    t = _ssp(jnp.dot(agg, c2w_ref[...], preferred_element_type=jnp.float32)
             + c2b_ref[...])
    h2 = (h1_ref[...]
          + jnp.dot(t, ilw_ref[...], preferred_element_type=jnp.float32)
          + ilb_ref[...])
    out_ref[...] = jnp.maximum(
        jnp.dot(h2, ow_ref[...], preferred_element_type=jnp.float32)
        + ob_ref[...], 0.0)


def _node_post_final(agg2, h1, c2w_t, c2b, ilw_t, ilb, ow_t, ob):
    grid = (N // _BN,)
    nb = N // _BN
    return pl.pallas_call(
        _node_post_final_body,
        grid=grid,
        in_specs=[
            pl.BlockSpec((_BN, NF), lambda i: (i, 0)),
            pl.BlockSpec((_BN, NF), lambda i, nb=nb: (i + nb, 0)),
            pl.BlockSpec((_BN, H), lambda i: (i, 0)),
            pl.BlockSpec((NF, H), lambda i: (0, 0)),
            pl.BlockSpec((1, H), lambda i: (0, 0)),
            pl.BlockSpec((H, H), lambda i: (0, 0)),
            pl.BlockSpec((1, H), lambda i: (0, 0)),
            pl.BlockSpec((H, NF), lambda i: (0, 0)),
            pl.BlockSpec((1, NF), lambda i: (0, 0)),
        ],
        out_specs=pl.BlockSpec((_BN, NF), lambda i: (i, 0)),
        out_shape=jax.ShapeDtypeStruct((N, NF), jnp.float32),
        compiler_params=pltpu.CompilerParams(
            dimension_semantics=("arbitrary",),
        ),
    )(agg2, agg2, h1, c2w_t, c2b, ilw_t, ilb, ow_t, ob)


_NC = 2
_NSUB = 16
_NW = _NC * _NSUB
_CH = 80
_NCHUNK = (E // _NW) // _CH
_NSLOT = 4
_TRIPS = _NCHUNK // _NSLOT + 1
_RPW = 624
_RREM = N - _NSUB * _RPW
assert _NCHUNK % _NSLOT != 0


def _make_edge_agg_body(col0):
    def _edge_agg_body(xl_hbm, we_hbm, src_hbm, dst_hbm, out_hbm,
                       idx_v, we_v, rows_v, acc_sh,
                       sidx, swe, sg, ss):
        cid = lax.axis_index("c")
        sid = lax.axis_index("s")
        wid = sid * _NC + cid
        c0 = wid * _NCHUNK

        def idx_cp(ci, s, d):
            hbm = dst_hbm if d else src_hbm
            return pltpu.make_async_copy(hbm.at[c0 + ci], idx_v.at[s, d],
                                         sidx[s])

        def we_cp(ci, s):
            return pltpu.make_async_copy(
                we_hbm.at[pl.ds((c0 + ci) * _CH, _CH), pl.ds(col0, NF)],
                we_v.at[s], swe[s])

        def gat_cp(s):
            return pltpu.make_async_copy(xl_hbm.at[idx_v.at[s, 0]],
                                         rows_v.at[s], sg[s])

        def sca_cp(s):
            return pltpu.make_async_copy(rows_v.at[s],
                                         acc_sh.at[idx_v.at[s, 1]], ss[s])

        def _zrow(i, _):
            for j in range(NF // 16):
                rows_v[0, i, pl.ds(j * 16, 16)] = jnp.zeros((16,), jnp.float32)
            return 0
        lax.fori_loop(0, _CH, _zrow, 0)
        base_r = sid * _RPW
        done = 0
        while done < _RPW:
            n = min(_CH, _RPW - done)
            pltpu.sync_copy(rows_v.at[0, pl.ds(0, n)],
                            acc_sh.at[pl.ds(base_r + done, n)])
            done += n

        @pl.when(sid == _NSUB - 1)
        def _():
            pltpu.sync_copy(rows_v.at[0, pl.ds(0, _RREM)],
                            acc_sh.at[pl.ds(_NSUB * _RPW, _RREM)])
        plsc.subcore_barrier()

        idx_cp(0, 0, 0).start()
        idx_cp(0, 0, 1).start()
        we_cp(0, 0).start()
        idx_cp(1, 1, 0).start()
        idx_cp(1, 1, 1).start()
        we_cp(1, 1).start()
        idx_cp(0, 0, 0).wait()
        idx_cp(0, 0, 1).wait()
        gat_cp(0).start()

        def _step(ci, s):
            s1 = (s + 1) % _NSLOT
            s2 = (s + 2) % _NSLOT

            @pl.when(jnp.logical_and(ci >= 2, ci <= _NCHUNK + 1))
            def _():
                sca_cp(s2).wait()

            @pl.when(ci + 2 < _NCHUNK)
            def _():
                idx_cp(ci + 2, s2, 0).start()
                idx_cp(ci + 2, s2, 1).start()
                we_cp(ci + 2, s2).start()

            @pl.when(ci + 1 < _NCHUNK)
            def _():
                idx_cp(ci + 1, s1, 0).wait()
                idx_cp(ci + 1, s1, 1).wait()
                gat_cp(s1).start()

            @pl.when(ci < _NCHUNK)
            def _():
                gat_cp(s).wait()
                we_cp(ci, s).wait()

                def _mul(i, _):
                    for r in range(4):
                        for j in range(NF // 16):
                            sl = pl.ds(j * 16, 16)
                            rows_v[s, 4 * i + r, sl] = (
                                rows_v[s, 4 * i + r, sl]
                                * we_v[s, 4 * i + r, sl])
                    return 0
                lax.fori_loop(0, _CH // 4, _mul, 0)
                sca_cp(s).start(add=True)

        def _quad(i, _):
            for k in range(_NSLOT):
                _step(_NSLOT * i + k, k)
            return 0
        lax.fori_loop(0, _TRIPS, _quad, 0)

        plsc.subcore_barrier()
        pltpu.sync_copy(acc_sh.at[pl.ds(base_r, _RPW)],
                        out_hbm.at[pl.ds(cid * N + base_r, _RPW)])

        @pl.when(sid == _NSUB - 1)
        def _():
            pltpu.sync_copy(acc_sh.at[pl.ds(_NSUB * _RPW, _RREM)],
                            out_hbm.at[pl.ds(cid * N + _NSUB * _RPW, _RREM)])

    return _edge_agg_body


@functools.cache
def _edge_agg_fn(col0):
    mesh = plsc.VectorSubcoreMesh(core_axis_name="c", subcore_axis_name="s")
    return pl.kernel(
        _make_edge_agg_body(col0),
        out_type=jax.ShapeDtypeStruct((_NC * N, NF), jnp.float32),
        mesh=mesh,
        scratch_types=[
            pltpu.VMEM((_NSLOT, 2, _CH), jnp.int32),
            pltpu.VMEM((_NSLOT, _CH, NF), jnp.float32),
            pltpu.VMEM((_NSLOT, _CH, NF), jnp.float32),
            pltpu.VMEM_SHARED((N, NF), jnp.float32),
            [pltpu.SemaphoreType.DMA] * _NSLOT,
            [pltpu.SemaphoreType.DMA] * _NSLOT,
            [pltpu.SemaphoreType.DMA] * _NSLOT,
            [pltpu.SemaphoreType.DMA] * _NSLOT,
        ],
        compiler_params=pltpu.CompilerParams(use_tc_tiling_on_sc=False),
    )


def _edge_agg(xl, we, src2, dst2, it):
    return _edge_agg_fn(it * NF)(xl, we, src2, dst2)


def kernel(h, edge_index, edge_weight, edge_attr, state_attr, batch,
           lin1_w_0, lin1_b_0, mlp1_w_0, mlp1_b_0, mlp2_w_0, mlp2_b_0,
           cf1_w_0, cf2_w_0, cf2_b_0, il_w_0, il_b_0,
           lin1_w_1, lin1_b_1, mlp1_w_1, mlp1_b_1, mlp2_w_1, mlp2_b_1,
           cf1_w_1, cf2_w_1, cf2_b_1, il_w_1, il_b_1, out_w, out_b):
    b2 = batch[:, None]
    ew_mat = edge_weight.reshape(E // 128, 128)
    src2 = edge_index[0].reshape(E // _CH, _CH)
    dst2 = edge_index[1].reshape(E // _CH, _CH)

    zblk = jnp.zeros((NF, NF), jnp.float32)
    m1cat = jnp.concatenate([mlp1_w_0.T, mlp1_w_1.T], axis=1)
    m1bcat = jnp.concatenate([mlp1_b_0, mlp1_b_1])[None, :]
    m2blk = jnp.block([[mlp2_w_0.T, zblk], [zblk, mlp2_w_1.T]])
    m2bcat = jnp.concatenate([mlp2_b_0, mlp2_b_1])[None, :]
    wec = _edge_filter(edge_attr, ew_mat, m1cat, m1bcat, m2blk, m2bcat)

    ph0 = jnp.pad(state_attr, ((0, 0), (0, H - NS)))
    pc0 = jnp.ones((G, 1), jnp.float32)

    h1, xl = _node_pre(h, ph0, pc0, b2,
                       lin1_w_0[:, NS:].T, lin1_w_0[:, :NS].T,
                       lin1_b_0[None, :], cf1_w_0.T)
    agg2 = _edge_agg(xl, wec, src2, dst2, 0)
    h2, ph, pc = _node_post_pool(agg2, h1, b2, cf2_w_0.T, cf2_b_0[None, :],
                                 il_w_0.T, il_b_0[None, :])

    h3, xl1 = _node_pre(h2, ph, pc, b2,
                        lin1_w_1[:, NS:].T, lin1_w_1[:, :NS].T,
                        lin1_b_1[None, :], cf1_w_1.T)
    agg2b = _edge_agg(xl1, wec, src2, dst2, 1)
    return _node_post_final(agg2b, h3, cf2_w_1.T, cf2_b_1[None, :],
                            il_w_1.T, il_b_1[None, :], out_w.T,
                            out_b[None, :])
